# baseline (speedup 1.0000x reference)
import jax
import jax.numpy as jnp
from jax.experimental import pallas as pl
from jax.experimental.pallas import tpu as pltpu

_N = 4096
_D = 2048
_TOPK = 20
_BR = 256
_BC = 256
_NEG = -3.0
_EPS = 1e-8


def _norm_kernel(x_ref, hi_ref, lo_ref):
    x = x_ref[...]
    n = jnp.sqrt(jnp.sum(x * x, axis=1, keepdims=True))
    xn = x / jnp.maximum(n, _EPS)
    hi = xn.astype(jnp.bfloat16)
    hi_ref[...] = hi
    lo_ref[...] = (xn - hi.astype(jnp.float32)).astype(jnp.bfloat16)


def _simtopk_kernel(ah_ref, al_ref, bh_ref, bl_ref, adj_ref, edge_ref, acc_ref):
    i = pl.program_id(0)
    j = pl.program_id(1)
    dims = (((1,), (1,)), ((), ()))
    ah, al = ah_ref[...], al_ref[...]
    bh, bl = bh_ref[...], bl_ref[...]
    sim = jax.lax.dot_general(ah, bh, dims, preferred_element_type=jnp.float32)
    sim += jax.lax.dot_general(ah, bl, dims, preferred_element_type=jnp.float32)
    sim += jax.lax.dot_general(al, bh, dims, preferred_element_type=jnp.float32)
    rows = jax.lax.broadcasted_iota(jnp.int32, sim.shape, 0)
    cols = jax.lax.broadcasted_iota(jnp.int32, sim.shape, 1)
    sim = jnp.where((i == j) & (rows == cols), 0.0, sim)
    acc_ref[:, pl.ds(j * _BC, _BC)] = sim

    @pl.when(j == _N // _BC - 1)
    def _():
        work = acc_ref[...]
        col = jax.lax.broadcasted_iota(jnp.int32, work.shape, 1)
        edge = jnp.zeros_like(work)
        for _ in range(_TOPK):
            m = jnp.max(work, axis=1, keepdims=True)
            cand = jnp.where(work == m, col, _N)
            amin = jnp.min(cand, axis=1, keepdims=True)
            sel = col == amin
            edge = jnp.where(sel, m, edge)
            work = jnp.where(sel, _NEG, work)
        edge_ref[...] = edge
        adj_ref[...] = jnp.maximum(edge, 0.0)


def kernel(h_style):
    hi, lo = pl.pallas_call(
        _norm_kernel,
        grid=(_N // _BR,),
        in_specs=[pl.BlockSpec((_BR, _D), lambda i: (i, 0))],
        out_specs=[
            pl.BlockSpec((_BR, _D), lambda i: (i, 0)),
            pl.BlockSpec((_BR, _D), lambda i: (i, 0)),
        ],
        out_shape=[
            jax.ShapeDtypeStruct((_N, _D), jnp.bfloat16),
            jax.ShapeDtypeStruct((_N, _D), jnp.bfloat16),
        ],
    )(h_style)

    adj, edge = pl.pallas_call(
        _simtopk_kernel,
        grid=(_N // _BR, _N // _BC),
        in_specs=[
            pl.BlockSpec((_BR, _D), lambda i, j: (i, 0)),
            pl.BlockSpec((_BR, _D), lambda i, j: (i, 0)),
            pl.BlockSpec((_BC, _D), lambda i, j: (j, 0)),
            pl.BlockSpec((_BC, _D), lambda i, j: (j, 0)),
        ],
        out_specs=[
            pl.BlockSpec((_BR, _N), lambda i, j: (i, 0)),
            pl.BlockSpec((_BR, _N), lambda i, j: (i, 0)),
        ],
        out_shape=[
            jax.ShapeDtypeStruct((_N, _N), jnp.float32),
            jax.ShapeDtypeStruct((_N, _N), jnp.float32),
        ],
        scratch_shapes=[pltpu.VMEM((_BR, _N), jnp.float32)],
        compiler_params=pltpu.CompilerParams(
            dimension_semantics=("arbitrary", "arbitrary")),
    )(hi, lo, hi, lo)
    return adj, edge[..., None]


# speedup vs baseline: 1.1744x; 1.1744x over previous
import jax
import jax.numpy as jnp
from jax.experimental import pallas as pl
from jax.experimental.pallas import tpu as pltpu

_N = 4096
_D = 2048
_TOPK = 20
_BR = 256
_BC = 256
_NEG = -3.0
_EPS = 1e-8


def _norm_kernel(x_ref, o_ref):
    x = x_ref[...]
    n = jnp.sqrt(jnp.sum(x * x, axis=1, keepdims=True))
    o_ref[...] = x / jnp.maximum(n, _EPS)


def _topk_iter(work_ref, edge_ref, col):
    work = work_ref[...]
    m = jnp.max(work, axis=1, keepdims=True)
    cand = jnp.where(work == m, col, _N)
    amin = jnp.min(cand, axis=1, keepdims=True)
    sel = col == amin
    edge_ref[...] = jnp.where(sel, m, edge_ref[...])
    work_ref[...] = jnp.where(sel, _NEG, work)


def _simtopk_kernel(a_ref, b_ref, adj_ref, edge_ref, acc0, acc1):
    i = pl.program_id(0)
    j = pl.program_id(1)
    ni = _N // _BR
    nj = _N // _BC
    base = _TOPK // nj
    extra = _TOPK % nj

    @pl.when(i < ni)
    def _():
        sim = jax.lax.dot_general(
            a_ref[...], b_ref[...], (((1,), (1,)), ((), ())),
            preferred_element_type=jnp.float32)
        rows = jax.lax.broadcasted_iota(jnp.int32, sim.shape, 0)
        cols = jax.lax.broadcasted_iota(jnp.int32, sim.shape, 1)
        sim = jnp.where((i == j) & (rows == cols), 0.0, sim)

        @pl.when(i % 2 == 0)
        def _():
            acc0[:, pl.ds(j * _BC, _BC)] = sim

        @pl.when(i % 2 == 1)
        def _():
            acc1[:, pl.ds(j * _BC, _BC)] = sim

    @pl.when(i > 0)
    def _():
        col = jax.lax.broadcasted_iota(jnp.int32, (_BR, _N), 1)

        @pl.when(j == 0)
        def _():
            edge_ref[...] = jnp.zeros((_BR, _N), jnp.float32)

        def run(work_ref):
            for _ in range(base):
                _topk_iter(work_ref, edge_ref, col)

            if extra:
                @pl.when(j < extra)
                def _():
                    _topk_iter(work_ref, edge_ref, col)

        @pl.when(i % 2 == 0)
        def _():
            run(acc1)

        @pl.when(i % 2 == 1)
        def _():
            run(acc0)

        @pl.when(j == nj - 1)
        def _():
            adj_ref[...] = jnp.maximum(edge_ref[...], 0.0)


def kernel(h_style):
    hn = pl.pallas_call(
        _norm_kernel,
        grid=(_N // _BR,),
        in_specs=[pl.BlockSpec((_BR, _D), lambda i: (i, 0))],
        out_specs=pl.BlockSpec((_BR, _D), lambda i: (i, 0)),
        out_shape=jax.ShapeDtypeStruct((_N, _D), jnp.float32),
    )(h_style)

    ni = _N // _BR
    adj, edge = pl.pallas_call(
        _simtopk_kernel,
        grid=(ni + 1, _N // _BC),
        in_specs=[
            pl.BlockSpec((_BR, _D), lambda i, j: (jnp.minimum(i, ni - 1), 0)),
            pl.BlockSpec((_BC, _D), lambda i, j: (j, 0)),
        ],
        out_specs=[
            pl.BlockSpec((_BR, _N), lambda i, j: (jnp.maximum(i, 1) - 1, 0)),
            pl.BlockSpec((_BR, _N), lambda i, j: (jnp.maximum(i, 1) - 1, 0)),
        ],
        out_shape=[
            jax.ShapeDtypeStruct((_N, _N), jnp.float32),
            jax.ShapeDtypeStruct((_N, _N), jnp.float32),
        ],
        scratch_shapes=[
            pltpu.VMEM((_BR, _N), jnp.float32),
            pltpu.VMEM((_BR, _N), jnp.float32),
        ],
        compiler_params=pltpu.CompilerParams(
            dimension_semantics=("arbitrary", "arbitrary")),
    )(hn, hn)
    return adj, edge[..., None]
